# Initial kernel scaffold; baseline (speedup 1.0000x reference)
import jax
import jax.numpy as jnp
from jax.experimental import pallas as pl


def kernel(x, edge_index, W1, b1, W2, b2):
    raise NotImplementedError("write your pallas kernel here")


# speedup vs baseline: 18.8143x; 18.8143x over previous
import functools

import jax
import jax.numpy as jnp
from jax import lax
from jax.experimental import pallas as pl
from jax.experimental.pallas import tpu as pltpu
from jax.experimental.pallas import tpu_sc as plsc

N = 10000
D = 128
H = 16
C = 40
E = 320000

NC = 2
NS = 16
NW = NC * NS
K = 128
CHUNKS = 79
EPW = CHUNKS * K
EPAD = EPW * NW
NACC = 10112
STRIPE = NACC // NS

_MESH = plsc.VectorSubcoreMesh(core_axis_name="c", subcore_axis_name="s")


def _sc_agg(width):
    @functools.partial(
        pl.kernel,
        out_type=jax.ShapeDtypeStruct((NC * NACC, width), jnp.float32),
        mesh=_MESH,
        scratch_types=[
            pltpu.VMEM((K,), jnp.int32),
            pltpu.VMEM((K,), jnp.int32),
            pltpu.VMEM((K, width), jnp.float32),
            pltpu.VMEM((STRIPE, width), jnp.float32),
            pltpu.SemaphoreType.DMA,
            pltpu.VMEM_SHARED((NACC, width), jnp.float32),
        ],
        compiler_params=pltpu.CompilerParams(use_tc_tiling_on_sc=False),
    )
    def agg(g_hbm, src_hbm, dst_hbm, zero_hbm, out_hbm, idx_s, idx_d, rows,
            stripe_v, sem, acc):
        cid = lax.axis_index("c")
        sid = lax.axis_index("s")
        wid = sid * NC + cid
        base = wid * EPW
        row0 = sid * STRIPE
        pltpu.sync_copy(zero_hbm.at[pl.ds(row0, STRIPE)], stripe_v)
        pltpu.sync_copy(stripe_v, acc.at[pl.ds(row0, STRIPE)])
        plsc.subcore_barrier()

        def body(ci, carry):
            off = base + ci * K
            pltpu.sync_copy(src_hbm.at[pl.ds(off, K)], idx_s)
            pltpu.sync_copy(dst_hbm.at[pl.ds(off, K)], idx_d)
            pltpu.async_copy(g_hbm.at[idx_s], rows, sem).wait()
            pltpu.sync_copy(rows, acc.at[idx_d], add=True)
            return carry

        lax.fori_loop(0, CHUNKS, body, 0)
        plsc.subcore_barrier()
        pltpu.sync_copy(acc.at[pl.ds(row0, STRIPE)], stripe_v)
        pltpu.sync_copy(stripe_v, out_hbm.at[pl.ds(cid * NACC + row0, STRIPE)])

    return agg


@functools.partial(
    pl.kernel,
    out_type=jax.ShapeDtypeStruct((NC * NACC,), jnp.float32),
    mesh=_MESH,
    scratch_types=[
        pltpu.VMEM((K,), jnp.int32),
        pltpu.VMEM((K,), jnp.float32),
        pltpu.VMEM((STRIPE,), jnp.float32),
        pltpu.VMEM_SHARED((NACC,), jnp.float32),
    ],
    compiler_params=pltpu.CompilerParams(use_tc_tiling_on_sc=False),
)
def _sc_deg(dst_hbm, zero_hbm, out_hbm, idx_d, ones, stripe_v, acc):
    cid = lax.axis_index("c")
    sid = lax.axis_index("s")
    wid = sid * NC + cid
    base = wid * EPW
    row0 = sid * STRIPE
    for i in range(K // 16):
        ones[pl.ds(i * 16, 16)] = jnp.full((16,), 1.0, jnp.float32)
    pltpu.sync_copy(zero_hbm.at[pl.ds(row0, STRIPE)], stripe_v)
    pltpu.sync_copy(stripe_v, acc.at[pl.ds(row0, STRIPE)])
    plsc.subcore_barrier()

    def body(ci, carry):
        off = base + ci * K
        pltpu.sync_copy(dst_hbm.at[pl.ds(off, K)], idx_d)
        pltpu.sync_copy(ones, acc.at[idx_d], add=True)
        return carry

    lax.fori_loop(0, CHUNKS, body, 0)
    plsc.subcore_barrier()
    pltpu.sync_copy(acc.at[pl.ds(row0, STRIPE)], stripe_v)
    pltpu.sync_copy(stripe_v, out_hbm.at[pl.ds(cid * NACC + row0, STRIPE)])


def _tc_a_body(degp_ref, x_ref, w1_ref, dinv_ref, g1_ref):
    deg = degp_ref[:, 0:1] + degp_ref[:, 1:2] + 1.0
    dinv = lax.rsqrt(deg)
    dinv_ref[...] = dinv
    h1 = jnp.dot(x_ref[...], w1_ref[...], preferred_element_type=jnp.float32)
    g1_ref[...] = h1 * dinv[:N]


def _tc_b_body(p1_ref, g1_ref, dinv_ref, b1_ref, w2_ref, g2_ref):
    s1 = p1_ref[0:N] + p1_ref[NACC:NACC + N] + g1_ref[...]
    dinv = dinv_ref[0:N]
    a1 = jnp.maximum(s1 * dinv + b1_ref[...], 0.0)
    h2 = jnp.dot(a1, w2_ref[...], preferred_element_type=jnp.float32)
    g2_ref[...] = h2 * dinv


def _tc_c_body(p2_ref, g2_ref, dinv_ref, b2_ref, out_ref):
    s2 = p2_ref[0:N] + p2_ref[NACC:NACC + N] + g2_ref[...]
    z = s2 * dinv_ref[0:N] + b2_ref[...]
    m = jnp.max(z, axis=1, keepdims=True)
    lse = jnp.log(jnp.sum(jnp.exp(z - m), axis=1, keepdims=True)) + m
    out_ref[...] = z - lse


_agg16 = _sc_agg(H)
_agg40 = _sc_agg(C)


def kernel(x, edge_index, W1, b1, W2, b2):
    pad = EPAD - E
    src_p = jnp.concatenate([edge_index[0], jnp.zeros((pad,), jnp.int32)])
    dst_p = jnp.concatenate([edge_index[1], jnp.full((pad,), N, jnp.int32)])
    z1 = jnp.zeros((NACC,), jnp.float32)
    zH = jnp.zeros((NACC, H), jnp.float32)
    zC = jnp.zeros((NACC, C), jnp.float32)

    degp = _sc_deg(dst_p, z1)

    dinv, g1 = pl.pallas_call(
        _tc_a_body,
        out_shape=(jax.ShapeDtypeStruct((NACC, 1), jnp.float32),
                   jax.ShapeDtypeStruct((N, H), jnp.float32)),
    )(degp.reshape(NC, NACC).T, x, W1)

    p1 = _agg16(g1, src_p, dst_p, zH)

    g2 = pl.pallas_call(
        _tc_b_body,
        out_shape=jax.ShapeDtypeStruct((N, C), jnp.float32),
    )(p1, g1, dinv, b1.reshape(1, H), W2)

    p2 = _agg40(g2, src_p, dst_p, zC)

    out = pl.pallas_call(
        _tc_c_body,
        out_shape=jax.ShapeDtypeStruct((N, C), jnp.float32),
    )(p2, g2, dinv, b2.reshape(1, C))
    return out


# speedup vs baseline: 28.0917x; 1.4931x over previous
import functools

import jax
import jax.numpy as jnp
from jax import lax
from jax.experimental import pallas as pl
from jax.experimental.pallas import tpu as pltpu
from jax.experimental.pallas import tpu_sc as plsc

N = 10000
D = 128
H = 16
C = 40
E = 320000

NC = 2
NS = 16
NW = NC * NS
K = 128
NB = 8
NG = 10
CHUNKS = NB * NG
EPW = CHUNKS * K
EPAD = EPW * NW
NACC = 10112
STRIPE = NACC // NS

_MESH = plsc.VectorSubcoreMesh(core_axis_name="c", subcore_axis_name="s")


def _sc_agg(width):
    @functools.partial(
        pl.kernel,
        out_type=jax.ShapeDtypeStruct((NC * NACC, width), jnp.float32),
        mesh=_MESH,
        scratch_types=[
            pltpu.VMEM((CHUNKS, K), jnp.int32),
            pltpu.VMEM((CHUNKS, K), jnp.int32),
            [pltpu.VMEM((K, width), jnp.float32) for _ in range(NB)],
            pltpu.VMEM((STRIPE, width), jnp.float32),
            pltpu.SemaphoreType.DMA,
            pltpu.SemaphoreType.DMA,
            pltpu.VMEM_SHARED((NACC, width), jnp.float32),
        ],
        compiler_params=pltpu.CompilerParams(use_tc_tiling_on_sc=False),
    )
    def agg(g_hbm, src_hbm, dst_hbm, zero_hbm, out_hbm, src_v, dst_v, rows,
            stripe_v, gsem, ssem, acc):
        cid = lax.axis_index("c")
        sid = lax.axis_index("s")
        wid = sid * NC + cid
        row0 = sid * STRIPE
        pltpu.async_copy(src_hbm.at[wid], src_v, gsem)
        pltpu.async_copy(dst_hbm.at[wid], dst_v, ssem)
        pltpu.sync_copy(zero_hbm.at[pl.ds(row0, STRIPE)], stripe_v)
        pltpu.sync_copy(stripe_v, acc.at[pl.ds(row0, STRIPE)])
        pltpu.make_async_copy(src_hbm.at[wid], src_v, gsem).wait()
        pltpu.make_async_copy(dst_hbm.at[wid], dst_v, ssem).wait()
        plsc.subcore_barrier()

        def body(g, carry):
            c0 = g * NB
            gd = [pltpu.async_copy(g_hbm.at[src_v.at[c0 + b]], rows[b], gsem)
                  for b in range(NB)]
            for d in gd:
                d.wait()
            sd = [pltpu.async_copy(rows[b], acc.at[dst_v.at[c0 + b]], ssem,
                                   add=True)
                  for b in range(NB)]
            for d in sd:
                d.wait()
            return carry

        lax.fori_loop(0, NG, body, 0)
        plsc.subcore_barrier()
        pltpu.sync_copy(acc.at[pl.ds(row0, STRIPE)], stripe_v)
        pltpu.sync_copy(stripe_v, out_hbm.at[pl.ds(cid * NACC + row0, STRIPE)])

    return agg


@functools.partial(
    pl.kernel,
    out_type=jax.ShapeDtypeStruct((NC * NACC,), jnp.float32),
    mesh=_MESH,
    scratch_types=[
        pltpu.VMEM((CHUNKS, K), jnp.int32),
        pltpu.VMEM((K,), jnp.float32),
        pltpu.VMEM((STRIPE,), jnp.float32),
        pltpu.SemaphoreType.DMA,
        pltpu.VMEM_SHARED((NACC,), jnp.float32),
    ],
    compiler_params=pltpu.CompilerParams(use_tc_tiling_on_sc=False),
)
def _sc_deg(dst_hbm, zero_hbm, out_hbm, dst_v, ones, stripe_v, sem, acc):
    cid = lax.axis_index("c")
    sid = lax.axis_index("s")
    wid = sid * NC + cid
    row0 = sid * STRIPE
    pltpu.async_copy(dst_hbm.at[wid], dst_v, sem)
    for i in range(K // 16):
        ones[pl.ds(i * 16, 16)] = jnp.full((16,), 1.0, jnp.float32)
    pltpu.sync_copy(zero_hbm.at[pl.ds(row0, STRIPE)], stripe_v)
    pltpu.sync_copy(stripe_v, acc.at[pl.ds(row0, STRIPE)])
    pltpu.make_async_copy(dst_hbm.at[wid], dst_v, sem).wait()
    plsc.subcore_barrier()

    def body(g, carry):
        c0 = g * NB
        sd = [pltpu.async_copy(ones, acc.at[dst_v.at[c0 + b]], sem, add=True)
              for b in range(NB)]
        for d in sd:
            d.wait()
        return carry

    lax.fori_loop(0, NG, body, 0)
    plsc.subcore_barrier()
    pltpu.sync_copy(acc.at[pl.ds(row0, STRIPE)], stripe_v)
    pltpu.sync_copy(stripe_v, out_hbm.at[pl.ds(cid * NACC + row0, STRIPE)])


def _tc_a_body(degp_ref, x_ref, w1_ref, dinv_ref, g1_ref):
    deg = degp_ref[:, 0:1] + degp_ref[:, 1:2] + 1.0
    dinv = lax.rsqrt(deg)
    dinv_ref[...] = dinv
    h1 = jnp.dot(x_ref[...], w1_ref[...], preferred_element_type=jnp.float32)
    g1_ref[...] = h1 * dinv[:N]


def _tc_b_body(p1_ref, g1_ref, dinv_ref, b1_ref, w2_ref, g2_ref):
    s1 = p1_ref[0:N] + p1_ref[NACC:NACC + N] + g1_ref[...]
    dinv = dinv_ref[0:N]
    a1 = jnp.maximum(s1 * dinv + b1_ref[...], 0.0)
    h2 = jnp.dot(a1, w2_ref[...], preferred_element_type=jnp.float32)
    g2_ref[...] = h2 * dinv


def _tc_c_body(p2_ref, g2_ref, dinv_ref, b2_ref, out_ref):
    s2 = p2_ref[0:N] + p2_ref[NACC:NACC + N] + g2_ref[...]
    z = s2 * dinv_ref[0:N] + b2_ref[...]
    m = jnp.max(z, axis=1, keepdims=True)
    lse = jnp.log(jnp.sum(jnp.exp(z - m), axis=1, keepdims=True)) + m
    out_ref[...] = z - lse


_agg16 = _sc_agg(H)
_agg40 = _sc_agg(C)


def kernel(x, edge_index, W1, b1, W2, b2):
    pad = EPAD - E
    src_p = jnp.concatenate([edge_index[0], jnp.zeros((pad,), jnp.int32)])
    pad_dst = N + jnp.arange(pad, dtype=jnp.int32) % (NACC - N)
    dst_p = jnp.concatenate([edge_index[1], pad_dst])
    src_p = src_p.reshape(NW, CHUNKS, K)
    dst_p = dst_p.reshape(NW, CHUNKS, K)
    z1 = jnp.zeros((NACC,), jnp.float32)
    zH = jnp.zeros((NACC, H), jnp.float32)
    zC = jnp.zeros((NACC, C), jnp.float32)

    degp = _sc_deg(dst_p, z1)

    dinv, g1 = pl.pallas_call(
        _tc_a_body,
        out_shape=(jax.ShapeDtypeStruct((NACC, 1), jnp.float32),
                   jax.ShapeDtypeStruct((N, H), jnp.float32)),
    )(degp.reshape(NC, NACC).T, x, W1)

    p1 = _agg16(g1, src_p, dst_p, zH)

    g2 = pl.pallas_call(
        _tc_b_body,
        out_shape=jax.ShapeDtypeStruct((N, C), jnp.float32),
    )(p1, g1, dinv, b1.reshape(1, H), W2)

    p2 = _agg40(g2, src_p, dst_p, zC)

    out = pl.pallas_call(
        _tc_c_body,
        out_shape=jax.ShapeDtypeStruct((N, C), jnp.float32),
    )(p2, g2, dinv, b2.reshape(1, C))
    return out


# speedup vs baseline: 29.0737x; 1.0350x over previous
import functools

import jax
import jax.numpy as jnp
from jax import lax
from jax.experimental import pallas as pl
from jax.experimental.pallas import tpu as pltpu
from jax.experimental.pallas import tpu_sc as plsc

N = 10000
D = 128
H = 16
C = 40
E = 320000

NC = 2
NS = 16
NW = NC * NS
K = 128
NB = 8
NG = 10
CHUNKS = NB * NG
EPW = CHUNKS * K
EPAD = EPW * NW
NACC = 10112
STRIPE = NACC // NS

_MESH = plsc.VectorSubcoreMesh(core_axis_name="c", subcore_axis_name="s")


def _sc_agg(width, NB, NG):
    HSTR = STRIPE // 2

    @functools.partial(
        pl.kernel,
        out_type=jax.ShapeDtypeStruct((NC * NACC, width), jnp.float32),
        mesh=_MESH,
        scratch_types=[
            pltpu.VMEM((CHUNKS, K), jnp.int32),
            pltpu.VMEM((CHUNKS, K), jnp.int32),
            [pltpu.VMEM((K, width), jnp.float32) for _ in range(NB)],
            [pltpu.VMEM((K, width), jnp.float32) for _ in range(NB)],
            pltpu.VMEM((HSTR, width), jnp.float32),
            pltpu.SemaphoreType.DMA,
            pltpu.SemaphoreType.DMA,
            pltpu.VMEM_SHARED((NACC, width), jnp.float32),
        ],
        compiler_params=pltpu.CompilerParams(use_tc_tiling_on_sc=False),
    )
    def agg(g_hbm, src_hbm, dst_hbm, zero_hbm, out_hbm, src_v, dst_v, rows_a,
            rows_b, half_v, gsem, ssem, acc):
        cid = lax.axis_index("c")
        sid = lax.axis_index("s")
        wid = sid * NC + cid
        row0 = sid * STRIPE

        def fire_g(bank, c0):
            for b in range(NB):
                pltpu.async_copy(g_hbm.at[src_v.at[c0 + b]], bank[b], gsem)

        def drain_g(bank, c0):
            for b in range(NB):
                pltpu.make_async_copy(g_hbm.at[src_v.at[c0 + b]], bank[b],
                                      gsem).wait()

        def fire_s(bank, c0):
            for b in range(NB):
                pltpu.async_copy(bank[b], acc.at[dst_v.at[c0 + b]], ssem,
                                 add=True)

        def drain_s(bank, c0):
            for b in range(NB):
                pltpu.make_async_copy(bank[b], acc.at[dst_v.at[c0 + b]],
                                      ssem).wait()

        pltpu.async_copy(src_hbm.at[wid], src_v, gsem)
        pltpu.async_copy(dst_hbm.at[wid], dst_v, ssem)
        for h in range(2):
            pltpu.sync_copy(zero_hbm.at[pl.ds(row0 + h * HSTR, HSTR)], half_v)
            pltpu.sync_copy(half_v, acc.at[pl.ds(row0 + h * HSTR, HSTR)])
        pltpu.make_async_copy(src_hbm.at[wid], src_v, gsem).wait()
        pltpu.make_async_copy(dst_hbm.at[wid], dst_v, ssem).wait()
        plsc.subcore_barrier()

        fire_g(rows_a, 0)

        def body(gg, carry):
            c0 = 2 * gg * NB
            drain_g(rows_a, c0)

            @pl.when(gg > 0)
            def _():
                drain_s(rows_b, c0 - NB)

            fire_g(rows_b, c0 + NB)
            fire_s(rows_a, c0)
            drain_g(rows_b, c0 + NB)
            drain_s(rows_a, c0)

            @pl.when(gg + 1 < NG // 2)
            def _():
                fire_g(rows_a, c0 + 2 * NB)

            fire_s(rows_b, c0 + NB)
            return carry

        lax.fori_loop(0, NG // 2, body, 0)
        drain_s(rows_b, CHUNKS - NB)
        plsc.subcore_barrier()
        for h in range(2):
            pltpu.sync_copy(acc.at[pl.ds(row0 + h * HSTR, HSTR)], half_v)
            pltpu.sync_copy(half_v,
                            out_hbm.at[pl.ds(cid * NACC + row0 + h * HSTR,
                                             HSTR)])

    return agg


@functools.partial(
    pl.kernel,
    out_type=jax.ShapeDtypeStruct((NC * NACC,), jnp.float32),
    mesh=_MESH,
    scratch_types=[
        pltpu.VMEM((CHUNKS, K), jnp.int32),
        pltpu.VMEM((K,), jnp.float32),
        pltpu.VMEM((STRIPE,), jnp.float32),
        pltpu.SemaphoreType.DMA,
        pltpu.VMEM_SHARED((NACC,), jnp.float32),
    ],
    compiler_params=pltpu.CompilerParams(use_tc_tiling_on_sc=False),
)
def _sc_deg(dst_hbm, zero_hbm, out_hbm, dst_v, ones, stripe_v, sem, acc):
    cid = lax.axis_index("c")
    sid = lax.axis_index("s")
    wid = sid * NC + cid
    row0 = sid * STRIPE
    pltpu.async_copy(dst_hbm.at[wid], dst_v, sem)
    for i in range(K // 16):
        ones[pl.ds(i * 16, 16)] = jnp.full((16,), 1.0, jnp.float32)
    pltpu.sync_copy(zero_hbm.at[pl.ds(row0, STRIPE)], stripe_v)
    pltpu.sync_copy(stripe_v, acc.at[pl.ds(row0, STRIPE)])
    pltpu.make_async_copy(dst_hbm.at[wid], dst_v, sem).wait()
    plsc.subcore_barrier()

    def body(g, carry):
        c0 = g * NB
        for b in range(NB):
            pltpu.async_copy(ones, acc.at[dst_v.at[c0 + b]], sem, add=True)

        @pl.when(g > 0)
        def _():
            for b in range(NB):
                pltpu.make_async_copy(ones, acc.at[dst_v.at[c0 - NB + b]],
                                      sem).wait()

        return carry

    lax.fori_loop(0, NG, body, 0)
    for b in range(NB):
        pltpu.make_async_copy(ones, acc.at[dst_v.at[CHUNKS - NB + b]],
                              sem).wait()
    plsc.subcore_barrier()
    pltpu.sync_copy(acc.at[pl.ds(row0, STRIPE)], stripe_v)
    pltpu.sync_copy(stripe_v, out_hbm.at[pl.ds(cid * NACC + row0, STRIPE)])


def _tc_a_body(degp_ref, x_ref, w1_ref, dinv_ref, g1_ref):
    deg = degp_ref[:, 0:1] + degp_ref[:, 1:2] + 1.0
    dinv = lax.rsqrt(deg)
    dinv_ref[...] = dinv
    h1 = jnp.dot(x_ref[...], w1_ref[...], preferred_element_type=jnp.float32)
    g1_ref[...] = h1 * dinv[:N]


def _tc_b_body(p1_ref, g1_ref, dinv_ref, b1_ref, w2_ref, g2_ref):
    s1 = p1_ref[0:N] + p1_ref[NACC:NACC + N] + g1_ref[...]
    dinv = dinv_ref[0:N]
    a1 = jnp.maximum(s1 * dinv + b1_ref[...], 0.0)
    h2 = jnp.dot(a1, w2_ref[...], preferred_element_type=jnp.float32)
    g2_ref[...] = h2 * dinv


def _tc_c_body(p2_ref, g2_ref, dinv_ref, b2_ref, out_ref):
    s2 = p2_ref[0:N] + p2_ref[NACC:NACC + N] + g2_ref[...]
    z = s2 * dinv_ref[0:N] + b2_ref[...]
    m = jnp.max(z, axis=1, keepdims=True)
    lse = jnp.log(jnp.sum(jnp.exp(z - m), axis=1, keepdims=True)) + m
    out_ref[...] = z - lse


_agg16 = _sc_agg(H, 8, 10)
_agg40 = _sc_agg(C, 4, 20)


def kernel(x, edge_index, W1, b1, W2, b2):
    pad = EPAD - E
    src_p = jnp.concatenate([edge_index[0], jnp.zeros((pad,), jnp.int32)])
    pad_dst = N + jnp.arange(pad, dtype=jnp.int32) % (NACC - N)
    dst_p = jnp.concatenate([edge_index[1], pad_dst])
    src_p = src_p.reshape(NW, CHUNKS, K)
    dst_p = dst_p.reshape(NW, CHUNKS, K)
    z1 = jnp.zeros((NACC,), jnp.float32)
    zH = jnp.zeros((NACC, H), jnp.float32)
    zC = jnp.zeros((NACC, C), jnp.float32)

    degp = _sc_deg(dst_p, z1)

    dinv, g1 = pl.pallas_call(
        _tc_a_body,
        out_shape=(jax.ShapeDtypeStruct((NACC, 1), jnp.float32),
                   jax.ShapeDtypeStruct((N, H), jnp.float32)),
    )(degp.reshape(NC, NACC).T, x, W1)

    p1 = _agg16(g1, src_p, dst_p, zH)

    g2 = pl.pallas_call(
        _tc_b_body,
        out_shape=jax.ShapeDtypeStruct((N, C), jnp.float32),
    )(p1, g1, dinv, b1.reshape(1, H), W2)

    p2 = _agg40(g2, src_p, dst_p, zC)

    out = pl.pallas_call(
        _tc_c_body,
        out_shape=jax.ShapeDtypeStruct((N, C), jnp.float32),
    )(p2, g2, dinv, b2.reshape(1, C))
    return out


# speedup vs baseline: 29.5292x; 1.0157x over previous
import functools

import jax
import jax.numpy as jnp
from jax import lax
from jax.experimental import pallas as pl
from jax.experimental.pallas import tpu as pltpu
from jax.experimental.pallas import tpu_sc as plsc

N = 10000
D = 128
H = 16
C = 40
E = 320000

NC = 2
NS = 16
NW = NC * NS
K = 128
NB = 8
NG = 10
CHUNKS = NB * NG
EPW = CHUNKS * K
EPAD = EPW * NW
NACC = 10112
STRIPE = NACC // NS

_MESH = plsc.VectorSubcoreMesh(core_axis_name="c", subcore_axis_name="s")


def _sc_agg(width, NB, NG):
    HSTR = STRIPE // 2

    @functools.partial(
        pl.kernel,
        out_type=jax.ShapeDtypeStruct((NC * NACC, width), jnp.float32),
        mesh=_MESH,
        scratch_types=[
            pltpu.VMEM((CHUNKS, K), jnp.int32),
            pltpu.VMEM((CHUNKS, K), jnp.int32),
            [pltpu.VMEM((K, width), jnp.float32) for _ in range(NB)],
            [pltpu.VMEM((K, width), jnp.float32) for _ in range(NB)],
            pltpu.VMEM((HSTR, width), jnp.float32),
            pltpu.SemaphoreType.DMA,
            pltpu.SemaphoreType.DMA,
            pltpu.VMEM_SHARED((NACC, width), jnp.float32),
        ],
        compiler_params=pltpu.CompilerParams(use_tc_tiling_on_sc=False),
    )
    def agg(g_hbm, src_hbm, dst_hbm, zero_hbm, out_hbm, src_v, dst_v, rows_a,
            rows_b, half_v, gsem, ssem, acc):
        cid = lax.axis_index("c")
        sid = lax.axis_index("s")
        wid = sid * NC + cid
        row0 = sid * STRIPE

        def fire_g(bank, c0):
            for b in range(NB):
                pltpu.async_copy(g_hbm.at[src_v.at[c0 + b]], bank[b], gsem)

        def drain_g(bank, c0):
            for b in range(NB):
                pltpu.make_async_copy(g_hbm.at[src_v.at[c0 + b]], bank[b],
                                      gsem).wait()

        def fire_s(bank, c0):
            for b in range(NB):
                pltpu.async_copy(bank[b], acc.at[dst_v.at[c0 + b]], ssem,
                                 add=True)

        def drain_s(bank, c0):
            for b in range(NB):
                pltpu.make_async_copy(bank[b], acc.at[dst_v.at[c0 + b]],
                                      ssem).wait()

        pltpu.async_copy(src_hbm.at[wid], src_v, gsem)
        pltpu.async_copy(dst_hbm.at[wid], dst_v, ssem)
        for h in range(2):
            pltpu.sync_copy(zero_hbm.at[pl.ds(row0 + h * HSTR, HSTR)], half_v)
            pltpu.sync_copy(half_v, acc.at[pl.ds(row0 + h * HSTR, HSTR)])
        pltpu.make_async_copy(src_hbm.at[wid], src_v, gsem).wait()
        pltpu.make_async_copy(dst_hbm.at[wid], dst_v, ssem).wait()
        plsc.subcore_barrier()

        fire_g(rows_a, 0)

        def body(gg, carry):
            c0 = 2 * gg * NB
            drain_g(rows_a, c0)

            @pl.when(gg > 0)
            def _():
                drain_s(rows_b, c0 - NB)

            fire_g(rows_b, c0 + NB)
            fire_s(rows_a, c0)
            drain_g(rows_b, c0 + NB)
            drain_s(rows_a, c0)

            @pl.when(gg + 1 < NG // 2)
            def _():
                fire_g(rows_a, c0 + 2 * NB)

            fire_s(rows_b, c0 + NB)
            return carry

        lax.fori_loop(0, NG // 2, body, 0)
        drain_s(rows_b, CHUNKS - NB)
        plsc.subcore_barrier()
        for h in range(2):
            pltpu.sync_copy(acc.at[pl.ds(row0 + h * HSTR, HSTR)], half_v)
            pltpu.sync_copy(half_v,
                            out_hbm.at[pl.ds(cid * NACC + row0 + h * HSTR,
                                             HSTR)])

    return agg


@functools.partial(
    pl.kernel,
    out_type=jax.ShapeDtypeStruct((NC * NACC,), jnp.float32),
    mesh=_MESH,
    scratch_types=[
        pltpu.VMEM((CHUNKS, K), jnp.int32),
        pltpu.VMEM((K,), jnp.float32),
        pltpu.VMEM((STRIPE,), jnp.float32),
        pltpu.SemaphoreType.DMA,
        pltpu.VMEM_SHARED((NACC,), jnp.float32),
    ],
    compiler_params=pltpu.CompilerParams(use_tc_tiling_on_sc=False),
)
def _sc_deg(dst_hbm, zero_hbm, out_hbm, dst_v, ones, stripe_v, sem, acc):
    cid = lax.axis_index("c")
    sid = lax.axis_index("s")
    wid = sid * NC + cid
    row0 = sid * STRIPE
    pltpu.async_copy(dst_hbm.at[wid], dst_v, sem)
    for i in range(K // 16):
        ones[pl.ds(i * 16, 16)] = jnp.full((16,), 1.0, jnp.float32)
    pltpu.sync_copy(zero_hbm.at[pl.ds(row0, STRIPE)], stripe_v)
    pltpu.sync_copy(stripe_v, acc.at[pl.ds(row0, STRIPE)])
    pltpu.make_async_copy(dst_hbm.at[wid], dst_v, sem).wait()
    plsc.subcore_barrier()

    def body(g, carry):
        c0 = g * NB
        for b in range(NB):
            pltpu.async_copy(ones, acc.at[dst_v.at[c0 + b]], sem, add=True)

        @pl.when(g > 0)
        def _():
            for b in range(NB):
                pltpu.make_async_copy(ones, acc.at[dst_v.at[c0 - NB + b]],
                                      sem).wait()

        return carry

    lax.fori_loop(0, NG, body, 0)
    for b in range(NB):
        pltpu.make_async_copy(ones, acc.at[dst_v.at[CHUNKS - NB + b]],
                              sem).wait()
    plsc.subcore_barrier()
    pltpu.sync_copy(acc.at[pl.ds(row0, STRIPE)], stripe_v)
    pltpu.sync_copy(stripe_v, out_hbm.at[pl.ds(cid * NACC + row0, STRIPE)])


def _tc_a_body(degp_ref, x_ref, w1_ref, dinv_ref, g1_ref):
    deg = degp_ref[:, 0:1] + degp_ref[:, 1:2] + 1.0
    dinv = lax.rsqrt(deg)
    dinv_ref[...] = dinv
    h1 = jnp.dot(x_ref[...], w1_ref[...], preferred_element_type=jnp.float32)
    g1_ref[...] = h1 * dinv[:N]


def _tc_b_body(p1_ref, g1_ref, dinv_ref, b1_ref, w2_ref, g2_ref):
    s1 = p1_ref[0:N] + p1_ref[NACC:NACC + N] + g1_ref[...]
    dinv = dinv_ref[0:N]
    a1 = jnp.maximum(s1 * dinv + b1_ref[...], 0.0)
    h2 = jnp.dot(a1, w2_ref[...], preferred_element_type=jnp.float32)
    g2_ref[...] = h2 * dinv


def _tc_c_body(p2_ref, g2_ref, dinv_ref, b2_ref, out_ref):
    s2 = p2_ref[0:N] + p2_ref[NACC:NACC + N] + g2_ref[...]
    z = s2 * dinv_ref[0:N] + b2_ref[...]
    m = jnp.max(z, axis=1, keepdims=True)
    lse = jnp.log(jnp.sum(jnp.exp(z - m), axis=1, keepdims=True)) + m
    out_ref[...] = z - lse


_agg16 = _sc_agg(H, 8, 10)
_agg40 = _sc_agg(C, 4, 20)


def kernel(x, edge_index, W1, b1, W2, b2):
    ppw = (EPAD - E) // NW
    src_r = edge_index[0].reshape(NW, E // NW)
    dst_r = edge_index[1].reshape(NW, E // NW)
    pad_src = jnp.zeros((NW, ppw), jnp.int32)
    pad_dst = jnp.broadcast_to(
        N + jnp.arange(ppw, dtype=jnp.int32) % (NACC - N), (NW, ppw))
    src_p = jnp.concatenate([src_r, pad_src], axis=1).reshape(NW, CHUNKS, K)
    dst_p = jnp.concatenate([dst_r, pad_dst], axis=1).reshape(NW, CHUNKS, K)
    z1 = jnp.zeros((NACC,), jnp.float32)
    zH = jnp.zeros((NACC, H), jnp.float32)
    zC = jnp.zeros((NACC, C), jnp.float32)

    degp = _sc_deg(dst_p, z1)

    dinv, g1 = pl.pallas_call(
        _tc_a_body,
        out_shape=(jax.ShapeDtypeStruct((NACC, 1), jnp.float32),
                   jax.ShapeDtypeStruct((N, H), jnp.float32)),
    )(degp.reshape(NC, NACC).T, x, W1)

    p1 = _agg16(g1, src_p, dst_p, zH)

    g2 = pl.pallas_call(
        _tc_b_body,
        out_shape=jax.ShapeDtypeStruct((N, C), jnp.float32),
    )(p1, g1, dinv, b1.reshape(1, H), W2)

    p2 = _agg40(g2, src_p, dst_p, zC)

    out = pl.pallas_call(
        _tc_c_body,
        out_shape=jax.ShapeDtypeStruct((N, C), jnp.float32),
    )(p2, g2, dinv, b2.reshape(1, C))
    return out


# speedup vs baseline: 42.3017x; 1.4325x over previous
import functools

import jax
import jax.numpy as jnp
from jax import lax
from jax.experimental import pallas as pl
from jax.experimental.pallas import tpu as pltpu
from jax.experimental.pallas import tpu_sc as plsc

N = 10000
D = 128
H = 16
C = 40
E = 320000

NC = 2
NS = 16
NW = NC * NS
K = 128
NB = 8
NG = 10
CHUNKS = NB * NG
EPW = CHUNKS * K
EPAD = EPW * NW
NACC = 10112
STRIPE = NACC // NS

_MESH = plsc.VectorSubcoreMesh(core_axis_name="c", subcore_axis_name="s")


def _sc_agg(width, NB, NG):
    HSTR = STRIPE // 2

    @functools.partial(
        pl.kernel,
        out_type=jax.ShapeDtypeStruct((NC * NACC, width), jnp.float32),
        mesh=_MESH,
        scratch_types=[
            pltpu.VMEM((CHUNKS, K), jnp.int32),
            pltpu.VMEM((CHUNKS, K), jnp.int32),
            [pltpu.VMEM((K, width), jnp.float32) for _ in range(NB)],
            [pltpu.VMEM((K, width), jnp.float32) for _ in range(NB)],
            pltpu.VMEM((HSTR, width), jnp.float32),
            pltpu.SemaphoreType.DMA,
            pltpu.SemaphoreType.DMA,
            pltpu.VMEM_SHARED((NACC, width), jnp.float32),
        ],
        compiler_params=pltpu.CompilerParams(use_tc_tiling_on_sc=False),
    )
    def agg(g_hbm, src_hbm, dst_hbm, zero_hbm, out_hbm, src_v, dst_v, rows_a,
            rows_b, half_v, gsem, ssem, acc):
        cid = lax.axis_index("c")
        sid = lax.axis_index("s")
        wid = sid * NC + cid
        row0 = sid * STRIPE

        def fire_g(bank, c0):
            for b in range(NB):
                pltpu.async_copy(g_hbm.at[src_v.at[c0 + b]], bank[b], gsem)

        def drain_g(bank, c0):
            for b in range(NB):
                pltpu.make_async_copy(g_hbm.at[src_v.at[c0 + b]], bank[b],
                                      gsem).wait()

        def fire_s(bank, c0):
            for b in range(NB):
                pltpu.async_copy(bank[b], acc.at[dst_v.at[c0 + b]], ssem,
                                 add=True)

        def drain_s(bank, c0):
            for b in range(NB):
                pltpu.make_async_copy(bank[b], acc.at[dst_v.at[c0 + b]],
                                      ssem).wait()

        pltpu.async_copy(src_hbm.at[wid], src_v, gsem)
        pltpu.async_copy(dst_hbm.at[wid], dst_v, ssem)
        for h in range(2):
            pltpu.sync_copy(zero_hbm.at[pl.ds(row0 + h * HSTR, HSTR)], half_v)
            pltpu.sync_copy(half_v, acc.at[pl.ds(row0 + h * HSTR, HSTR)])
        pltpu.make_async_copy(src_hbm.at[wid], src_v, gsem).wait()
        pltpu.make_async_copy(dst_hbm.at[wid], dst_v, ssem).wait()
        plsc.subcore_barrier()

        fire_g(rows_a, 0)

        def body(gg, carry):
            c0 = 2 * gg * NB
            drain_g(rows_a, c0)

            @pl.when(gg > 0)
            def _():
                drain_s(rows_b, c0 - NB)

            fire_g(rows_b, c0 + NB)
            fire_s(rows_a, c0)
            drain_g(rows_b, c0 + NB)
            drain_s(rows_a, c0)

            @pl.when(gg + 1 < NG // 2)
            def _():
                fire_g(rows_a, c0 + 2 * NB)

            fire_s(rows_b, c0 + NB)
            return carry

        lax.fori_loop(0, NG // 2, body, 0)
        drain_s(rows_b, CHUNKS - NB)
        plsc.subcore_barrier()
        for h in range(2):
            pltpu.sync_copy(acc.at[pl.ds(row0 + h * HSTR, HSTR)], half_v)
            pltpu.sync_copy(half_v,
                            out_hbm.at[pl.ds(cid * NACC + row0 + h * HSTR,
                                             HSTR)])

    return agg


@functools.partial(
    pl.kernel,
    out_type=jax.ShapeDtypeStruct((NC * NACC,), jnp.float32),
    mesh=_MESH,
    scratch_types=[
        pltpu.VMEM((CHUNKS, K), jnp.int32),
        pltpu.VMEM((K,), jnp.float32),
        pltpu.VMEM((STRIPE,), jnp.float32),
        pltpu.SemaphoreType.DMA,
        pltpu.VMEM_SHARED((NACC,), jnp.float32),
    ],
    compiler_params=pltpu.CompilerParams(use_tc_tiling_on_sc=False),
)
def _sc_deg(dst_hbm, zero_hbm, out_hbm, dst_v, ones, stripe_v, sem, acc):
    cid = lax.axis_index("c")
    sid = lax.axis_index("s")
    wid = sid * NC + cid
    row0 = sid * STRIPE
    pltpu.async_copy(dst_hbm.at[wid], dst_v, sem)
    for i in range(K // 16):
        ones[pl.ds(i * 16, 16)] = jnp.full((16,), 1.0, jnp.float32)
    pltpu.sync_copy(zero_hbm.at[pl.ds(row0, STRIPE)], stripe_v)
    pltpu.sync_copy(stripe_v, acc.at[pl.ds(row0, STRIPE)])
    pltpu.make_async_copy(dst_hbm.at[wid], dst_v, sem).wait()
    plsc.subcore_barrier()

    def body(g, carry):
        c0 = g * NB
        for b in range(NB):
            pltpu.async_copy(ones, acc.at[dst_v.at[c0 + b]], sem, add=True)

        @pl.when(g > 0)
        def _():
            for b in range(NB):
                pltpu.make_async_copy(ones, acc.at[dst_v.at[c0 - NB + b]],
                                      sem).wait()

        return carry

    lax.fori_loop(0, NG, body, 0)
    for b in range(NB):
        pltpu.make_async_copy(ones, acc.at[dst_v.at[CHUNKS - NB + b]],
                              sem).wait()
    plsc.subcore_barrier()
    pltpu.sync_copy(acc.at[pl.ds(row0, STRIPE)], stripe_v)
    pltpu.sync_copy(stripe_v, out_hbm.at[pl.ds(cid * NACC + row0, STRIPE)])


def _tc_a_body(degp_ref, x_ref, w1_ref, dinv_ref, g1_ref):
    deg = degp_ref[:, 0:1] + degp_ref[:, 1:2] + 1.0
    dinv = lax.rsqrt(deg)
    dinv_ref[...] = dinv
    h1 = jnp.dot(x_ref[...], w1_ref[...], preferred_element_type=jnp.float32)
    g1_ref[...] = h1 * dinv[:N]


def _tc_b_body(p1_ref, g1_ref, dinv_ref, b1_ref, ga1_ref):
    s1 = p1_ref[0:N] + p1_ref[NACC:NACC + N] + g1_ref[...]
    dinv = dinv_ref[0:N]
    a1 = jnp.maximum(s1 * dinv + b1_ref[...], 0.0)
    ga1_ref[...] = a1 * dinv


def _tc_c_body(p2_ref, ga1_ref, dinv_ref, w2_ref, b2_ref, out_ref):
    u = p2_ref[0:N] + p2_ref[NACC:NACC + N] + ga1_ref[...]
    s2 = u * dinv_ref[0:N]
    z = jnp.dot(s2, w2_ref[...], preferred_element_type=jnp.float32) \
        + b2_ref[...]
    m = jnp.max(z, axis=1, keepdims=True)
    lse = jnp.log(jnp.sum(jnp.exp(z - m), axis=1, keepdims=True)) + m
    out_ref[...] = z - lse


_agg16 = _sc_agg(H, 8, 10)


def kernel(x, edge_index, W1, b1, W2, b2):
    ppw = (EPAD - E) // NW
    src_r = edge_index[0].reshape(NW, E // NW)
    dst_r = edge_index[1].reshape(NW, E // NW)
    pad_src = jnp.zeros((NW, ppw), jnp.int32)
    pad_dst = jnp.broadcast_to(
        N + jnp.arange(ppw, dtype=jnp.int32) % (NACC - N), (NW, ppw))
    src_p = jnp.concatenate([src_r, pad_src], axis=1).reshape(NW, CHUNKS, K)
    dst_p = jnp.concatenate([dst_r, pad_dst], axis=1).reshape(NW, CHUNKS, K)
    z1 = jnp.zeros((NACC,), jnp.float32)
    zH = jnp.zeros((NACC, H), jnp.float32)

    degp = _sc_deg(dst_p, z1)

    dinv, g1 = pl.pallas_call(
        _tc_a_body,
        out_shape=(jax.ShapeDtypeStruct((NACC, 1), jnp.float32),
                   jax.ShapeDtypeStruct((N, H), jnp.float32)),
    )(degp.reshape(NC, NACC).T, x, W1)

    p1 = _agg16(g1, src_p, dst_p, zH)

    ga1 = pl.pallas_call(
        _tc_b_body,
        out_shape=jax.ShapeDtypeStruct((N, H), jnp.float32),
    )(p1, g1, dinv, b1.reshape(1, H))

    p2 = _agg16(ga1, src_p, dst_p, zH)

    out = pl.pallas_call(
        _tc_c_body,
        out_shape=jax.ShapeDtypeStruct((N, C), jnp.float32),
    )(p2, ga1, dinv, W2, b2.reshape(1, C))
    return out


# speedup vs baseline: 49.9333x; 1.1804x over previous
import functools

import jax
import jax.numpy as jnp
from jax import lax
from jax.experimental import pallas as pl
from jax.experimental.pallas import tpu as pltpu
from jax.experimental.pallas import tpu_sc as plsc

N = 10000
D = 128
H = 16
C = 40
E = 320000

NC = 2
NS = 16
NW = NC * NS
K = 128
NB = 8
NG = 10
CHUNKS = NB * NG
EPW = CHUNKS * K
EPAD = EPW * NW
NACC = 10112
STRIPE = NACC // NS

SEG = 128 // H
RH = N * H // 128
PH = NACC * H // 128

_MESH = plsc.VectorSubcoreMesh(core_axis_name="c", subcore_axis_name="s")


def _sc_agg(width, NB, NG):
    HSTR = STRIPE // 2

    @functools.partial(
        pl.kernel,
        out_type=jax.ShapeDtypeStruct((NC * NACC, width), jnp.float32),
        mesh=_MESH,
        scratch_types=[
            pltpu.VMEM((CHUNKS, K), jnp.int32),
            pltpu.VMEM((CHUNKS, K), jnp.int32),
            [pltpu.VMEM((K, width), jnp.float32) for _ in range(NB)],
            [pltpu.VMEM((K, width), jnp.float32) for _ in range(NB)],
            pltpu.VMEM((HSTR, width), jnp.float32),
            pltpu.SemaphoreType.DMA,
            pltpu.SemaphoreType.DMA,
            pltpu.VMEM_SHARED((NACC, width), jnp.float32),
        ],
        compiler_params=pltpu.CompilerParams(use_tc_tiling_on_sc=False),
    )
    def agg(g_hbm, src_hbm, dst_hbm, zero_hbm, out_hbm, src_v, dst_v, rows_a,
            rows_b, half_v, gsem, ssem, acc):
        cid = lax.axis_index("c")
        sid = lax.axis_index("s")
        wid = sid * NC + cid
        row0 = sid * STRIPE

        def fire_g(bank, c0):
            for b in range(NB):
                pltpu.async_copy(g_hbm.at[src_v.at[c0 + b]], bank[b], gsem)

        def drain_g(bank, c0):
            for b in range(NB):
                pltpu.make_async_copy(g_hbm.at[src_v.at[c0 + b]], bank[b],
                                      gsem).wait()

        def fire_s(bank, c0):
            for b in range(NB):
                pltpu.async_copy(bank[b], acc.at[dst_v.at[c0 + b]], ssem,
                                 add=True)

        def drain_s(bank, c0):
            for b in range(NB):
                pltpu.make_async_copy(bank[b], acc.at[dst_v.at[c0 + b]],
                                      ssem).wait()

        pltpu.async_copy(src_hbm.at[wid], src_v, gsem)
        pltpu.async_copy(dst_hbm.at[wid], dst_v, ssem)
        for h in range(2):
            pltpu.sync_copy(zero_hbm.at[pl.ds(row0 + h * HSTR, HSTR)], half_v)
            pltpu.sync_copy(half_v, acc.at[pl.ds(row0 + h * HSTR, HSTR)])
        pltpu.make_async_copy(src_hbm.at[wid], src_v, gsem).wait()
        pltpu.make_async_copy(dst_hbm.at[wid], dst_v, ssem).wait()
        plsc.subcore_barrier()

        fire_g(rows_a, 0)

        def body(gg, carry):
            c0 = 2 * gg * NB
            drain_g(rows_a, c0)

            @pl.when(gg > 0)
            def _():
                drain_s(rows_b, c0 - NB)

            fire_g(rows_b, c0 + NB)
            fire_s(rows_a, c0)
            drain_g(rows_b, c0 + NB)
            drain_s(rows_a, c0)

            @pl.when(gg + 1 < NG // 2)
            def _():
                fire_g(rows_a, c0 + 2 * NB)

            fire_s(rows_b, c0 + NB)
            return carry

        lax.fori_loop(0, NG // 2, body, 0)
        drain_s(rows_b, CHUNKS - NB)
        plsc.subcore_barrier()
        for h in range(2):
            pltpu.sync_copy(acc.at[pl.ds(row0 + h * HSTR, HSTR)], half_v)
            pltpu.sync_copy(half_v,
                            out_hbm.at[pl.ds(cid * NACC + row0 + h * HSTR,
                                             HSTR)])

    return agg


@functools.partial(
    pl.kernel,
    out_type=jax.ShapeDtypeStruct((NC * NACC,), jnp.float32),
    mesh=_MESH,
    scratch_types=[
        pltpu.VMEM((CHUNKS, K), jnp.int32),
        pltpu.VMEM((K,), jnp.float32),
        pltpu.VMEM((STRIPE,), jnp.float32),
        pltpu.SemaphoreType.DMA,
        pltpu.VMEM_SHARED((NACC,), jnp.float32),
    ],
    compiler_params=pltpu.CompilerParams(use_tc_tiling_on_sc=False),
)
def _sc_deg(dst_hbm, zero_hbm, out_hbm, dst_v, ones, stripe_v, sem, acc):
    cid = lax.axis_index("c")
    sid = lax.axis_index("s")
    wid = sid * NC + cid
    row0 = sid * STRIPE
    pltpu.async_copy(dst_hbm.at[wid], dst_v, sem)
    for i in range(K // 16):
        ones[pl.ds(i * 16, 16)] = jnp.full((16,), 1.0, jnp.float32)
    pltpu.sync_copy(zero_hbm.at[pl.ds(row0, STRIPE)], stripe_v)
    pltpu.sync_copy(stripe_v, acc.at[pl.ds(row0, STRIPE)])
    pltpu.make_async_copy(dst_hbm.at[wid], dst_v, sem).wait()
    plsc.subcore_barrier()

    def body(g, carry):
        c0 = g * NB
        for b in range(NB):
            pltpu.async_copy(ones, acc.at[dst_v.at[c0 + b]], sem, add=True)

        @pl.when(g > 0)
        def _():
            for b in range(NB):
                pltpu.make_async_copy(ones, acc.at[dst_v.at[c0 - NB + b]],
                                      sem).wait()

        return carry

    lax.fori_loop(0, NG, body, 0)
    for b in range(NB):
        pltpu.make_async_copy(ones, acc.at[dst_v.at[CHUNKS - NB + b]],
                              sem).wait()
    plsc.subcore_barrier()
    pltpu.sync_copy(acc.at[pl.ds(row0, STRIPE)], stripe_v)
    pltpu.sync_copy(stripe_v, out_hbm.at[pl.ds(cid * NACC + row0, STRIPE)])


def _tc_a_body(degr_ref, xr_ref, w1b_ref, dinvr_ref, g1r_ref):
    degr = degr_ref[0:RH] + degr_ref[PH:PH + RH] + 1.0
    dinvr = lax.rsqrt(degr)
    dinvr_ref[...] = dinvr
    h1r = jnp.dot(xr_ref[...], w1b_ref[...],
                  preferred_element_type=jnp.float32)
    g1r_ref[...] = h1r * dinvr


def _tc_b_body(p1r_ref, g1r_ref, dinvr_ref, b1b_ref, ga1r_ref):
    s1r = p1r_ref[0:RH] + p1r_ref[PH:PH + RH] + g1r_ref[...]
    dinvr = dinvr_ref[...]
    a1r = jnp.maximum(s1r * dinvr + b1b_ref[...], 0.0)
    ga1r_ref[...] = a1r * dinvr


def _tc_c_body(p2r_ref, ga1r_ref, dinvr_ref, w2b_ref, b2b_ref, out_ref):
    ur = p2r_ref[0:RH] + p2r_ref[PH:PH + RH] + ga1r_ref[...]
    s2r = ur * dinvr_ref[...]
    zr = jnp.dot(s2r, w2b_ref[...], preferred_element_type=jnp.float32) \
        + b2b_ref[...]
    for j in range(SEG):
        zs = zr[:, C * j:C * (j + 1)]
        m = jnp.max(zs, axis=1, keepdims=True)
        lse = jnp.log(jnp.sum(jnp.exp(zs - m), axis=1, keepdims=True)) + m
        out_ref[:, C * j:C * (j + 1)] = zs - lse


_agg16 = _sc_agg(H, 8, 10)


def kernel(x, edge_index, W1, b1, W2, b2):
    ppw = (EPAD - E) // NW
    src_r = edge_index[0].reshape(NW, E // NW)
    dst_r = edge_index[1].reshape(NW, E // NW)
    pad_src = jnp.zeros((NW, ppw), jnp.int32)
    pad_dst = jnp.broadcast_to(
        N + jnp.arange(ppw, dtype=jnp.int32) % (NACC - N), (NW, ppw))
    src_p = jnp.concatenate([src_r, pad_src], axis=1).reshape(NW, CHUNKS, K)
    dst_p = jnp.concatenate([dst_r, pad_dst], axis=1).reshape(NW, CHUNKS, K)
    z1 = jnp.zeros((NACC,), jnp.float32)
    zH = jnp.zeros((NACC, H), jnp.float32)

    degp = _sc_deg(dst_p, z1)
    degr = jnp.broadcast_to(degp[:, None],
                            (NC * NACC, H)).reshape(NC * PH, 128)
    w1b = jnp.kron(jnp.eye(SEG, dtype=jnp.float32), W1)

    dinvr, g1r = pl.pallas_call(
        _tc_a_body,
        out_shape=(jax.ShapeDtypeStruct((RH, 128), jnp.float32),
                   jax.ShapeDtypeStruct((RH, 128), jnp.float32)),
    )(degr, x.reshape(RH, SEG * D), w1b)

    p1 = _agg16(g1r.reshape(N, H), src_p, dst_p, zH)

    ga1r = pl.pallas_call(
        _tc_b_body,
        out_shape=jax.ShapeDtypeStruct((RH, 128), jnp.float32),
    )(p1.reshape(NC * PH, 128), g1r, dinvr,
      jnp.tile(b1, SEG).reshape(1, 128))

    p2 = _agg16(ga1r.reshape(N, H), src_p, dst_p, zH)

    w2b = jnp.kron(jnp.eye(SEG, dtype=jnp.float32), W2)
    outr = pl.pallas_call(
        _tc_c_body,
        out_shape=jax.ShapeDtypeStruct((RH, SEG * C), jnp.float32),
    )(p2.reshape(NC * PH, 128), ga1r, dinvr, w2b,
      jnp.tile(b2, SEG).reshape(1, SEG * C))
    return outr.reshape(N, C)
